# baseline (speedup 1.0000x reference)
import functools

import jax
import jax.numpy as jnp
from jax import lax
from jax.experimental import pallas as pl
from jax.experimental.pallas import tpu as pltpu
from jax.experimental.pallas import tpu_sc as plsc

N = 10000
E = 320000
D_IN = 128
H = 16
D_E = 4
L = 3
OUT = 2

NC = 2
NS = 16
NW = NC * NS
EW = E // NW
CHUNK = 1000
NCHUNK = EW // CHUNK
UROW = (D_E + 1) * H
NPAD = 10240
ROWS_PER_TILE = NPAD // NS


def _in_proj_kernel(x_ref, w_ref, b_ref, wcat_ref, rw_ref, rb_ref,
                    u_ref, hr_ref):
    h0 = jnp.dot(x_ref[...], w_ref[...],
                 preferred_element_type=jnp.float32,
                 precision=lax.Precision.HIGHEST) + b_ref[...]
    u_ref[...] = jnp.dot(h0, wcat_ref[...], preferred_element_type=jnp.float32,
                 precision=lax.Precision.HIGHEST)
    hr_ref[...] = jnp.dot(h0, rw_ref[...],
                          preferred_element_type=jnp.float32,
                 precision=lax.Precision.HIGHEST) + rb_ref[...]


def _mid_kernel(hr_ref, part_ref, g_ref, b_ref, wcat_ref, rw_ref, rb_ref,
                u_ref, hro_ref):
    t = (hr_ref[...] + part_ref[0, pl.ds(0, N), :]
         + part_ref[1, pl.ds(0, N), :])
    mean = jnp.mean(t, axis=0, keepdims=True)
    var = jnp.mean((t - mean) ** 2, axis=0, keepdims=True)
    hn = (t - mean) * lax.rsqrt(var + 1e-5) * g_ref[...] + b_ref[...]
    hn = jnp.maximum(hn, 0.0)
    u_ref[...] = jnp.dot(hn, wcat_ref[...], preferred_element_type=jnp.float32,
                 precision=lax.Precision.HIGHEST)
    hro_ref[...] = jnp.dot(hn, rw_ref[...],
                           preferred_element_type=jnp.float32,
                 precision=lax.Precision.HIGHEST) + rb_ref[...]


def _final_kernel(hr_ref, part_ref, g_ref, b_ref, cw_ref, cb_ref, o_ref):
    t = (hr_ref[...] + part_ref[0, pl.ds(0, N), :]
         + part_ref[1, pl.ds(0, N), :])
    mean = jnp.mean(t, axis=0, keepdims=True)
    var = jnp.mean((t - mean) ** 2, axis=0, keepdims=True)
    hn = (t - mean) * lax.rsqrt(var + 1e-5) * g_ref[...] + b_ref[...]
    hn = jnp.maximum(hn, 0.0)
    o_ref[...] = jnp.dot(hn, cw_ref[...],
                         preferred_element_type=jnp.float32,
                 precision=lax.Precision.HIGHEST) + cb_ref[...]


_in_proj = pl.pallas_call(
    _in_proj_kernel,
    out_shape=(jax.ShapeDtypeStruct((N, UROW), jnp.float32),
               jax.ShapeDtypeStruct((N, H), jnp.float32)),
)

_mid = pl.pallas_call(
    _mid_kernel,
    out_shape=(jax.ShapeDtypeStruct((N, UROW), jnp.float32),
               jax.ShapeDtypeStruct((N, H), jnp.float32)),
)

_final = pl.pallas_call(
    _final_kernel,
    out_shape=jax.ShapeDtypeStruct((N, OUT), jnp.float32),
)


def _edge_body(u_hbm, src_hbm, dst_hbm, ea_hbm, out_hbm,
               srcv, dstv, eav, rows, msg, buf, agg, sem):
    cid = lax.axis_index("c")
    sid = lax.axis_index("s")

    zero = jnp.zeros((H,), jnp.float32)

    def _zbody(i, c):
        buf[i, :] = zero
        return c
    lax.fori_loop(0, ROWS_PER_TILE, _zbody, 0, unroll=8)

    pltpu.sync_copy(buf, agg.at[pl.ds(sid * ROWS_PER_TILE, ROWS_PER_TILE)])
    plsc.subcore_barrier()

    wid = sid * NC + cid
    for k in range(NCHUNK):
        base = wid * EW + k * CHUNK
        pltpu.sync_copy(src_hbm.at[pl.ds(base, CHUNK)], srcv)
        pltpu.sync_copy(dst_hbm.at[pl.ds(base, CHUNK)], dstv)
        pltpu.sync_copy(ea_hbm.at[pl.ds(base * D_E, CHUNK * D_E)],
                        eav.at[pl.ds(0, CHUNK * D_E)])
        pltpu.async_copy(u_hbm.at[srcv], rows, sem).wait()

        def _ebody(e, c):
            a = eav[pl.ds(D_E * e, H)]
            m = rows[e, pl.ds(4 * H, H)]
            m = m + a[0] * rows[e, pl.ds(0, H)]
            m = m + a[1] * rows[e, pl.ds(H, H)]
            m = m + a[2] * rows[e, pl.ds(2 * H, H)]
            m = m + a[3] * rows[e, pl.ds(3 * H, H)]
            msg[e, :] = m
            return c
        lax.fori_loop(0, CHUNK, _ebody, 0, unroll=4)

        pltpu.sync_copy(msg, agg.at[dstv], add=True)

    plsc.subcore_barrier()
    pltpu.sync_copy(agg.at[pl.ds(sid * ROWS_PER_TILE, ROWS_PER_TILE)], buf)
    pltpu.sync_copy(buf,
                    out_hbm.at[cid, pl.ds(sid * ROWS_PER_TILE,
                                          ROWS_PER_TILE)])


_edge_pass = pl.kernel(
    _edge_body,
    out_type=jax.ShapeDtypeStruct((NC, NPAD, H), jnp.float32),
    mesh=plsc.VectorSubcoreMesh(core_axis_name="c", subcore_axis_name="s",
                                num_cores=NC, num_subcores=NS),
    compiler_params=pltpu.CompilerParams(use_tc_tiling_on_sc=False),
    scratch_types=[
        pltpu.VMEM((CHUNK,), jnp.int32),
        pltpu.VMEM((CHUNK,), jnp.int32),
        pltpu.VMEM((CHUNK * D_E + H,), jnp.float32),
        pltpu.VMEM((CHUNK, UROW), jnp.float32),
        pltpu.VMEM((CHUNK, H), jnp.float32),
        pltpu.VMEM((ROWS_PER_TILE, H), jnp.float32),
        pltpu.VMEM_SHARED((NPAD, H), jnp.float32),
        pltpu.SemaphoreType.DMA,
    ],
)


def kernel(x, edge_index, edge_attr, lin_in_w, lin_in_b, enn_w, enn_b,
           root_w, root_b, bn_g, bn_b, cls_w, cls_b):
    src = edge_index[0]
    dst = edge_index[1]
    ea_flat = edge_attr.reshape(-1)

    wcat = jnp.concatenate(
        [enn_w.reshape(L, D_E, H, H).transpose(0, 2, 1, 3).reshape(L, H, D_E * H),
         enn_b.reshape(L, H, H)], axis=2)

    u, hr = _in_proj(x, lin_in_w, lin_in_b.reshape(1, H),
                     wcat[0], root_w[0], root_b[0].reshape(1, H))
    for l in range(L):
        part = _edge_pass(u, src, dst, ea_flat)
        if l + 1 < L:
            u, hr = _mid(hr, part, bn_g[l].reshape(1, H), bn_b[l].reshape(1, H),
                         wcat[l + 1], root_w[l + 1], root_b[l + 1].reshape(1, H))
        else:
            out = _final(hr, part, bn_g[l].reshape(1, H), bn_b[l].reshape(1, H),
                         cls_w, cls_b.reshape(1, OUT))
    return out


# speedup vs baseline: 1.2347x; 1.2347x over previous
import functools

import jax
import jax.numpy as jnp
from jax import lax
from jax.experimental import pallas as pl
from jax.experimental.pallas import tpu as pltpu
from jax.experimental.pallas import tpu_sc as plsc

N = 10000
E = 320000
D_IN = 128
H = 16
D_E = 4
L = 3
OUT = 2

NC = 2
NS = 16
NW = NC * NS
EW = E // NW
PAIR = 1024
LASTP = EW - 9 * PAIR
NPAIRS = 10
CH = PAIR // 2
LASTH = LASTP // 2
NH = NPAIRS * 2
_PLEN = [PAIR] * 9 + [LASTP]
_HLEN = [CH] * 18 + [LASTH] * 2
UROW = (D_E + 1) * H
NPAD = 10112
ROWS_PER_TILE = NPAD // NS

_PREC = lax.Precision.HIGHEST


def _in_proj_kernel(x_ref, w_ref, b_ref, wcat_ref, rw_ref, rb_ref,
                    u_ref, hr_ref):
    h0 = jnp.dot(x_ref[...], w_ref[...], preferred_element_type=jnp.float32,
                 precision=_PREC) + b_ref[...]
    u_ref[...] = jnp.dot(h0, wcat_ref[...],
                         preferred_element_type=jnp.float32, precision=_PREC)
    hr_ref[...] = jnp.dot(h0, rw_ref[...], preferred_element_type=jnp.float32,
                          precision=_PREC) + rb_ref[...]


def _mid_kernel(hr_ref, part_ref, g_ref, b_ref, wcat_ref, rw_ref, rb_ref,
                u_ref, hro_ref):
    t = (hr_ref[...] + part_ref[0, pl.ds(0, N), :]
         + part_ref[1, pl.ds(0, N), :])
    mean = jnp.mean(t, axis=0, keepdims=True)
    var = jnp.mean((t - mean) ** 2, axis=0, keepdims=True)
    hn = (t - mean) * lax.rsqrt(var + 1e-5) * g_ref[...] + b_ref[...]
    hn = jnp.maximum(hn, 0.0)
    u_ref[...] = jnp.dot(hn, wcat_ref[...],
                         preferred_element_type=jnp.float32, precision=_PREC)
    hro_ref[...] = jnp.dot(hn, rw_ref[...], preferred_element_type=jnp.float32,
                           precision=_PREC) + rb_ref[...]


def _final_kernel(hr_ref, part_ref, g_ref, b_ref, cw_ref, cb_ref, o_ref):
    t = (hr_ref[...] + part_ref[0, pl.ds(0, N), :]
         + part_ref[1, pl.ds(0, N), :])
    mean = jnp.mean(t, axis=0, keepdims=True)
    var = jnp.mean((t - mean) ** 2, axis=0, keepdims=True)
    hn = (t - mean) * lax.rsqrt(var + 1e-5) * g_ref[...] + b_ref[...]
    hn = jnp.maximum(hn, 0.0)
    o_ref[...] = jnp.dot(hn, cw_ref[...], preferred_element_type=jnp.float32,
                         precision=_PREC) + cb_ref[...]


_in_proj = pl.pallas_call(
    _in_proj_kernel,
    out_shape=(jax.ShapeDtypeStruct((N, UROW), jnp.float32),
               jax.ShapeDtypeStruct((N, H), jnp.float32)),
)

_mid = pl.pallas_call(
    _mid_kernel,
    out_shape=(jax.ShapeDtypeStruct((N, UROW), jnp.float32),
               jax.ShapeDtypeStruct((N, H), jnp.float32)),
)

_final = pl.pallas_call(
    _final_kernel,
    out_shape=jax.ShapeDtypeStruct((N, OUT), jnp.float32),
)


def _edge_body(u_hbm, src_hbm, dst_hbm, ea_hbm, out_hbm,
               src2, dst2, ea2, rows, msg, buf, agg,
               isem, gsem, ssem):
    cid = lax.axis_index("c")
    sid = lax.axis_index("s")

    zero = jnp.zeros((H,), jnp.float32)

    def _zbody(i, c):
        buf[i, :] = zero
        return c
    lax.fori_loop(0, ROWS_PER_TILE, _zbody, 0, unroll=8)

    pltpu.sync_copy(buf, agg.at[pl.ds(sid * ROWS_PER_TILE, ROWS_PER_TILE)])
    plsc.subcore_barrier()

    wid = sid * NC + cid
    wbase = wid * EW

    def stage(p):
        b = p % 2
        base = wbase + p * PAIR
        n = _PLEN[p]
        return [
            pltpu.async_copy(src_hbm.at[pl.ds(base, n)],
                             src2.at[b, pl.ds(0, n)], isem),
            pltpu.async_copy(dst_hbm.at[pl.ds(base, n)],
                             dst2.at[b, pl.ds(0, n)], isem),
            pltpu.async_copy(ea_hbm.at[pl.ds(base * D_E, n * D_E)],
                             ea2.at[b, pl.ds(0, n * D_E)], isem),
        ]

    def gather(h):
        p, hh, hb, n = h // 2, h % 2, h % 2, _HLEN[h]
        return pltpu.async_copy(
            u_hbm.at[src2.at[p % 2, pl.ds(hh * n, n)]],
            rows.at[hb, pl.ds(0, n)], gsem)

    def compute(h):
        hb = h % 2
        p, hh, b, n = h // 2, h % 2, (h // 2) % 2, _HLEN[h]
        ea_off = hh * n * D_E

        def _ebody(e, c):
            a = ea2[b, pl.ds(ea_off + D_E * e, H)]
            m = rows[hb, e, pl.ds(4 * H, H)]
            m = m + a[0] * rows[hb, e, pl.ds(0, H)]
            m = m + a[1] * rows[hb, e, pl.ds(H, H)]
            m = m + a[2] * rows[hb, e, pl.ds(2 * H, H)]
            m = m + a[3] * rows[hb, e, pl.ds(3 * H, H)]
            msg[hb, e, :] = m
            return c
        lax.fori_loop(0, n, _ebody, 0, unroll=4)

    def scatter(h):
        hb = h % 2
        p, hh, b, n = h // 2, h % 2, (h // 2) % 2, _HLEN[h]
        idx = dst2.at[b, pl.ds(hh * n, n)]
        return pltpu.async_copy(msg.at[hb, pl.ds(0, n)], agg.at[idx], ssem,
                                add=True)

    st = {0: stage(0)}
    for d in st[0]:
        d.wait()
    gd = {0: gather(0)}
    sc = {}
    for h in range(NH):
        p = h // 2
        if h >= 1:
            sc[h - 1].wait()
        if h + 1 < NH:
            if (h + 1) % 2 == 1:
                gd[h + 1] = gather(h + 1)
                if p + 1 < NPAIRS:
                    st[p + 1] = stage(p + 1)
            else:
                for d in st[p + 1]:
                    d.wait()
                gd[h + 1] = gather(h + 1)
        gd[h].wait()
        compute(h)
        sc[h] = scatter(h)
    sc[NH - 1].wait()

    plsc.subcore_barrier()
    pltpu.sync_copy(agg.at[pl.ds(sid * ROWS_PER_TILE, ROWS_PER_TILE)], buf)
    pltpu.sync_copy(buf,
                    out_hbm.at[cid, pl.ds(sid * ROWS_PER_TILE,
                                          ROWS_PER_TILE)])


_edge_pass = pl.kernel(
    _edge_body,
    out_type=jax.ShapeDtypeStruct((NC, NPAD, H), jnp.float32),
    mesh=plsc.VectorSubcoreMesh(core_axis_name="c", subcore_axis_name="s",
                                num_cores=NC, num_subcores=NS),
    compiler_params=pltpu.CompilerParams(use_tc_tiling_on_sc=False),
    scratch_types=[
        pltpu.VMEM((2, PAIR), jnp.int32),
        pltpu.VMEM((2, PAIR), jnp.int32),
        pltpu.VMEM((2, PAIR * D_E + H), jnp.float32),
        pltpu.VMEM((2, CH, UROW), jnp.float32),
        pltpu.VMEM((2, CH, H), jnp.float32),
        pltpu.VMEM((ROWS_PER_TILE, H), jnp.float32),
        pltpu.VMEM_SHARED((NPAD, H), jnp.float32),
        pltpu.SemaphoreType.DMA,
        pltpu.SemaphoreType.DMA,
        pltpu.SemaphoreType.DMA,
    ],
)


def kernel(x, edge_index, edge_attr, lin_in_w, lin_in_b, enn_w, enn_b,
           root_w, root_b, bn_g, bn_b, cls_w, cls_b):
    src = edge_index[0]
    dst = edge_index[1]
    ea_flat = edge_attr.reshape(-1)

    wcat = jnp.concatenate(
        [enn_w.reshape(L, D_E, H, H).transpose(0, 2, 1, 3).reshape(L, H, D_E * H),
         enn_b.reshape(L, H, H)], axis=2)

    u, hr = _in_proj(x, lin_in_w, lin_in_b.reshape(1, H),
                     wcat[0], root_w[0], root_b[0].reshape(1, H))
    for l in range(L):
        part = _edge_pass(u, src, dst, ea_flat)
        if l + 1 < L:
            u, hr = _mid(hr, part, bn_g[l].reshape(1, H), bn_b[l].reshape(1, H),
                         wcat[l + 1], root_w[l + 1], root_b[l + 1].reshape(1, H))
        else:
            out = _final(hr, part, bn_g[l].reshape(1, H), bn_b[l].reshape(1, H),
                         cls_w, cls_b.reshape(1, OUT))
    return out


# speedup vs baseline: 1.3514x; 1.0945x over previous
import functools

import jax
import jax.numpy as jnp
from jax import lax
from jax.experimental import pallas as pl
from jax.experimental.pallas import tpu as pltpu
from jax.experimental.pallas import tpu_sc as plsc

N = 10000
E = 320000
D_IN = 128
H = 16
D_E = 4
L = 3
OUT = 2

NC = 2
NS = 16
NW = NC * NS
EW = E // NW
PAIR = 1024
LASTP = EW - 9 * PAIR
NPAIRS = 10
CH = PAIR // 2
LASTH = LASTP // 2
NH = NPAIRS * 2
_PLEN = [PAIR] * 9 + [LASTP]
_HLEN = [CH] * 18 + [LASTH] * 2
UROW = (D_E + 1) * H
NPAD = 10112
ROWS_PER_TILE = NPAD // NS

_PREC = lax.Precision.HIGHEST


def _in_proj_kernel(x_ref, w_ref, b_ref, wcat_ref, rw_ref, rb_ref,
                    u_ref, hr_ref):
    h0 = jnp.dot(x_ref[...], w_ref[...], preferred_element_type=jnp.float32,
                 precision=_PREC) + b_ref[...]
    u_ref[...] = jnp.dot(h0, wcat_ref[...],
                         preferred_element_type=jnp.float32, precision=_PREC)
    hr_ref[...] = jnp.dot(h0, rw_ref[...], preferred_element_type=jnp.float32,
                          precision=_PREC) + rb_ref[...]


def _mid_kernel(hr_ref, part_ref, g_ref, b_ref, wcat_ref, rw_ref, rb_ref,
                u_ref, hro_ref):
    t = (hr_ref[...] + part_ref[0, pl.ds(0, N), :]
         + part_ref[1, pl.ds(0, N), :])
    mean = jnp.mean(t, axis=0, keepdims=True)
    var = jnp.mean((t - mean) ** 2, axis=0, keepdims=True)
    hn = (t - mean) * lax.rsqrt(var + 1e-5) * g_ref[...] + b_ref[...]
    hn = jnp.maximum(hn, 0.0)
    u_ref[...] = jnp.dot(hn, wcat_ref[...],
                         preferred_element_type=jnp.float32, precision=_PREC)
    hro_ref[...] = jnp.dot(hn, rw_ref[...], preferred_element_type=jnp.float32,
                           precision=_PREC) + rb_ref[...]


def _final_kernel(hr_ref, part_ref, g_ref, b_ref, cw_ref, cb_ref, o_ref):
    t = (hr_ref[...] + part_ref[0, pl.ds(0, N), :]
         + part_ref[1, pl.ds(0, N), :])
    mean = jnp.mean(t, axis=0, keepdims=True)
    var = jnp.mean((t - mean) ** 2, axis=0, keepdims=True)
    hn = (t - mean) * lax.rsqrt(var + 1e-5) * g_ref[...] + b_ref[...]
    hn = jnp.maximum(hn, 0.0)
    o_ref[...] = jnp.dot(hn, cw_ref[...], preferred_element_type=jnp.float32,
                         precision=_PREC) + cb_ref[...]


_in_proj = pl.pallas_call(
    _in_proj_kernel,
    out_shape=(jax.ShapeDtypeStruct((N, UROW), jnp.float32),
               jax.ShapeDtypeStruct((N, H), jnp.float32)),
)

_mid = pl.pallas_call(
    _mid_kernel,
    out_shape=(jax.ShapeDtypeStruct((N, UROW), jnp.float32),
               jax.ShapeDtypeStruct((N, H), jnp.float32)),
)

_final = pl.pallas_call(
    _final_kernel,
    out_shape=jax.ShapeDtypeStruct((N, OUT), jnp.float32),
)


PCH = 1280
NPCH = E // PCH
SUB = 256


def _prep_body(ei_hbm, ea_hbm, src_out, dst_out, eaf_out,
               idxb, eab, outb, sem):
    cid = lax.axis_index("c")
    sid = lax.axis_index("s")
    wid = sid * NC + cid
    iot = lax.iota(jnp.int32, 16)
    ri0 = lax.shift_right_logical(iot, 2)
    li = lax.bitwise_and(iot, 3)
    for k in range(8):
        c = wid + NW * k

        @pl.when(c < NPCH)
        def _():
            base = c * PCH
            pltpu.sync_copy(ei_hbm.at[0, pl.ds(base, PCH)], idxb)
            pltpu.sync_copy(idxb, src_out.at[pl.ds(base, PCH)])
            pltpu.sync_copy(ei_hbm.at[1, pl.ds(base, PCH)], idxb)
            pltpu.sync_copy(idxb, dst_out.at[pl.ds(base, PCH)])
            for j in range(PCH // SUB):
                pltpu.sync_copy(ea_hbm.at[pl.ds(base + j * SUB, SUB)], eab)

                def _gbody(g, carry, _j=j):
                    v = plsc.load_gather(eab, [ri0 + 4 * g, li])
                    outb[pl.ds(_j * SUB * D_E + 16 * g, 16)] = v
                    return carry
                lax.fori_loop(0, SUB // 4, _gbody, 0, unroll=4)
            pltpu.sync_copy(outb, eaf_out.at[pl.ds(base * D_E, PCH * D_E)])


_edge_prep = pl.kernel(
    _prep_body,
    out_type=(jax.ShapeDtypeStruct((E,), jnp.int32),
              jax.ShapeDtypeStruct((E,), jnp.int32),
              jax.ShapeDtypeStruct((E * D_E,), jnp.float32)),
    mesh=plsc.VectorSubcoreMesh(core_axis_name="c", subcore_axis_name="s",
                                num_cores=NC, num_subcores=NS),
    compiler_params=pltpu.CompilerParams(needs_layout_passes=False),
    scratch_types=[
        pltpu.VMEM((PCH,), jnp.int32),
        pltpu.VMEM((SUB, D_E), jnp.float32),
        pltpu.VMEM((PCH * D_E,), jnp.float32),
        pltpu.SemaphoreType.DMA,
    ],
)


def _edge_body(u_hbm, src_hbm, dst_hbm, ea_hbm, out_hbm,
               src2, dst2, ea2, rows, msg, buf, agg,
               isem, gsem, ssem):
    cid = lax.axis_index("c")
    sid = lax.axis_index("s")

    zero = jnp.zeros((H,), jnp.float32)

    def _zbody(i, c):
        buf[i, :] = zero
        return c
    lax.fori_loop(0, ROWS_PER_TILE, _zbody, 0, unroll=8)

    pltpu.sync_copy(buf, agg.at[pl.ds(sid * ROWS_PER_TILE, ROWS_PER_TILE)])
    plsc.subcore_barrier()

    wid = sid * NC + cid
    wbase = wid * EW

    def stage(p):
        b = p % 2
        base = wbase + p * PAIR
        n = _PLEN[p]
        return [
            pltpu.async_copy(src_hbm.at[pl.ds(base, n)],
                             src2.at[b, pl.ds(0, n)], isem),
            pltpu.async_copy(dst_hbm.at[pl.ds(base, n)],
                             dst2.at[b, pl.ds(0, n)], isem),
            pltpu.async_copy(ea_hbm.at[pl.ds(base * D_E, n * D_E)],
                             ea2.at[b, pl.ds(0, n * D_E)], isem),
        ]

    def gather(h):
        p, hh, hb, n = h // 2, h % 2, h % 2, _HLEN[h]
        return pltpu.async_copy(
            u_hbm.at[src2.at[p % 2, pl.ds(hh * n, n)]],
            rows.at[hb, pl.ds(0, n)], gsem)

    def compute(h):
        hb = h % 2
        p, hh, b, n = h // 2, h % 2, (h // 2) % 2, _HLEN[h]
        ea_off = hh * n * D_E

        def _ebody(e, c):
            a = ea2[b, pl.ds(ea_off + D_E * e, H)]
            m = rows[hb, e, pl.ds(4 * H, H)]
            m = m + a[0] * rows[hb, e, pl.ds(0, H)]
            m = m + a[1] * rows[hb, e, pl.ds(H, H)]
            m = m + a[2] * rows[hb, e, pl.ds(2 * H, H)]
            m = m + a[3] * rows[hb, e, pl.ds(3 * H, H)]
            msg[hb, e, :] = m
            return c
        lax.fori_loop(0, n, _ebody, 0, unroll=4)

    def scatter(h):
        hb = h % 2
        p, hh, b, n = h // 2, h % 2, (h // 2) % 2, _HLEN[h]
        idx = dst2.at[b, pl.ds(hh * n, n)]
        return pltpu.async_copy(msg.at[hb, pl.ds(0, n)], agg.at[idx], ssem,
                                add=True)

    st = {0: stage(0)}
    for d in st[0]:
        d.wait()
    gd = {0: gather(0)}
    sc = {}
    for h in range(NH):
        p = h // 2
        if h >= 1:
            sc[h - 1].wait()
        if h + 1 < NH:
            if (h + 1) % 2 == 1:
                gd[h + 1] = gather(h + 1)
                if p + 1 < NPAIRS:
                    st[p + 1] = stage(p + 1)
            else:
                for d in st[p + 1]:
                    d.wait()
                gd[h + 1] = gather(h + 1)
        gd[h].wait()
        compute(h)
        sc[h] = scatter(h)
    sc[NH - 1].wait()

    plsc.subcore_barrier()
    pltpu.sync_copy(agg.at[pl.ds(sid * ROWS_PER_TILE, ROWS_PER_TILE)], buf)
    pltpu.sync_copy(buf,
                    out_hbm.at[cid, pl.ds(sid * ROWS_PER_TILE,
                                          ROWS_PER_TILE)])


_edge_pass = pl.kernel(
    _edge_body,
    out_type=jax.ShapeDtypeStruct((NC, NPAD, H), jnp.float32),
    mesh=plsc.VectorSubcoreMesh(core_axis_name="c", subcore_axis_name="s",
                                num_cores=NC, num_subcores=NS),
    compiler_params=pltpu.CompilerParams(use_tc_tiling_on_sc=False),
    scratch_types=[
        pltpu.VMEM((2, PAIR), jnp.int32),
        pltpu.VMEM((2, PAIR), jnp.int32),
        pltpu.VMEM((2, PAIR * D_E + H), jnp.float32),
        pltpu.VMEM((2, CH, UROW), jnp.float32),
        pltpu.VMEM((2, CH, H), jnp.float32),
        pltpu.VMEM((ROWS_PER_TILE, H), jnp.float32),
        pltpu.VMEM_SHARED((NPAD, H), jnp.float32),
        pltpu.SemaphoreType.DMA,
        pltpu.SemaphoreType.DMA,
        pltpu.SemaphoreType.DMA,
    ],
)


def kernel(x, edge_index, edge_attr, lin_in_w, lin_in_b, enn_w, enn_b,
           root_w, root_b, bn_g, bn_b, cls_w, cls_b):
    src, dst, ea_flat = _edge_prep(edge_index, edge_attr)

    wcat = jnp.concatenate(
        [enn_w.reshape(L, D_E, H, H).transpose(0, 2, 1, 3).reshape(L, H, D_E * H),
         enn_b.reshape(L, H, H)], axis=2)

    u, hr = _in_proj(x, lin_in_w, lin_in_b.reshape(1, H),
                     wcat[0], root_w[0], root_b[0].reshape(1, H))
    for l in range(L):
        part = _edge_pass(u, src, dst, ea_flat)
        if l + 1 < L:
            u, hr = _mid(hr, part, bn_g[l].reshape(1, H), bn_b[l].reshape(1, H),
                         wcat[l + 1], root_w[l + 1], root_b[l + 1].reshape(1, H))
        else:
            out = _final(hr, part, bn_g[l].reshape(1, H), bn_b[l].reshape(1, H),
                         cls_w, cls_b.reshape(1, OUT))
    return out


# speedup vs baseline: 1.3889x; 1.0278x over previous
import functools

import jax
import jax.numpy as jnp
from jax import lax
from jax.experimental import pallas as pl
from jax.experimental.pallas import tpu as pltpu
from jax.experimental.pallas import tpu_sc as plsc

N = 10000
E = 320000
D_IN = 128
H = 16
D_E = 4
L = 3
OUT = 2

NC = 2
NS = 16
NW = NC * NS
EW = E // NW
PAIR = 1024
LASTP = EW - 9 * PAIR
NPAIRS = 10
CH = PAIR // 2
LASTH = LASTP // 2
NH = NPAIRS * 2
_PLEN = [PAIR] * 9 + [LASTP]
_HLEN = [CH] * 18 + [LASTH] * 2
UROW = (D_E + 1) * H
NPAD = 10112
ROWS_PER_TILE = NPAD // NS

_PREC = lax.Precision.HIGHEST


def _in_proj_kernel(x_ref, w_ref, b_ref, wcat_ref, rw_ref, rb_ref,
                    u_ref, hr_ref):
    h0 = jnp.dot(x_ref[...], w_ref[...], preferred_element_type=jnp.float32,
                 precision=_PREC) + b_ref[...]
    u_ref[...] = jnp.dot(h0, wcat_ref[...],
                         preferred_element_type=jnp.float32, precision=_PREC)
    hr_ref[...] = jnp.dot(h0, rw_ref[...], preferred_element_type=jnp.float32,
                          precision=_PREC) + rb_ref[...]


def _mid_kernel(hr_ref, part_ref, g_ref, b_ref, wcat_ref, rw_ref, rb_ref,
                u_ref, hro_ref):
    t = (hr_ref[...] + part_ref[0, pl.ds(0, N), :]
         + part_ref[1, pl.ds(0, N), :])
    mean = jnp.mean(t, axis=0, keepdims=True)
    var = jnp.mean((t - mean) ** 2, axis=0, keepdims=True)
    hn = (t - mean) * lax.rsqrt(var + 1e-5) * g_ref[...] + b_ref[...]
    hn = jnp.maximum(hn, 0.0)
    u_ref[...] = jnp.dot(hn, wcat_ref[...],
                         preferred_element_type=jnp.float32, precision=_PREC)
    hro_ref[...] = jnp.dot(hn, rw_ref[...], preferred_element_type=jnp.float32,
                           precision=_PREC) + rb_ref[...]


def _final_kernel(hr_ref, part_ref, g_ref, b_ref, cw_ref, cb_ref, o_ref):
    t = (hr_ref[...] + part_ref[0, pl.ds(0, N), :]
         + part_ref[1, pl.ds(0, N), :])
    mean = jnp.mean(t, axis=0, keepdims=True)
    var = jnp.mean((t - mean) ** 2, axis=0, keepdims=True)
    hn = (t - mean) * lax.rsqrt(var + 1e-5) * g_ref[...] + b_ref[...]
    hn = jnp.maximum(hn, 0.0)
    o_ref[...] = jnp.dot(hn, cw_ref[...], preferred_element_type=jnp.float32,
                         precision=_PREC) + cb_ref[...]


_in_proj = pl.pallas_call(
    _in_proj_kernel,
    out_shape=(jax.ShapeDtypeStruct((N, UROW), jnp.float32),
               jax.ShapeDtypeStruct((N, H), jnp.float32)),
)

_mid = pl.pallas_call(
    _mid_kernel,
    out_shape=(jax.ShapeDtypeStruct((N, UROW), jnp.float32),
               jax.ShapeDtypeStruct((N, H), jnp.float32)),
)

_final = pl.pallas_call(
    _final_kernel,
    out_shape=jax.ShapeDtypeStruct((N, OUT), jnp.float32),
)


PCH = 1280
NPCH = E // PCH
SUB = 256


def _prep_body(ei_hbm, ea_hbm, src_out, dst_out, eaf_out,
               idxb, eab, outb, sem):
    cid = lax.axis_index("c")
    sid = lax.axis_index("s")
    wid = sid * NC + cid
    iot = lax.iota(jnp.int32, 16)
    ri0 = lax.shift_right_logical(iot, 2)
    li = lax.bitwise_and(iot, 3)
    for k in range(8):
        c = wid + NW * k

        @pl.when(c < NPCH)
        def _():
            base = c * PCH
            pltpu.sync_copy(ei_hbm.at[0, pl.ds(base, PCH)], idxb)
            pltpu.sync_copy(idxb, src_out.at[pl.ds(base, PCH)])
            pltpu.sync_copy(ei_hbm.at[1, pl.ds(base, PCH)], idxb)
            pltpu.sync_copy(idxb, dst_out.at[pl.ds(base, PCH)])
            for j in range(PCH // SUB):
                pltpu.sync_copy(ea_hbm.at[pl.ds(base + j * SUB, SUB)], eab)

                def _gbody(g, carry, _j=j):
                    v = plsc.load_gather(eab, [ri0 + 4 * g, li])
                    outb[pl.ds(_j * SUB * D_E + 16 * g, 16)] = v
                    return carry
                lax.fori_loop(0, SUB // 4, _gbody, 0, unroll=4)
            pltpu.sync_copy(outb, eaf_out.at[pl.ds(base * D_E, PCH * D_E)])


_edge_prep = pl.kernel(
    _prep_body,
    out_type=(jax.ShapeDtypeStruct((E,), jnp.int32),
              jax.ShapeDtypeStruct((E,), jnp.int32),
              jax.ShapeDtypeStruct((E * D_E,), jnp.float32)),
    mesh=plsc.VectorSubcoreMesh(core_axis_name="c", subcore_axis_name="s",
                                num_cores=NC, num_subcores=NS),
    compiler_params=pltpu.CompilerParams(needs_layout_passes=False),
    scratch_types=[
        pltpu.VMEM((PCH,), jnp.int32),
        pltpu.VMEM((SUB, D_E), jnp.float32),
        pltpu.VMEM((PCH * D_E,), jnp.float32),
        pltpu.SemaphoreType.DMA,
    ],
)


def _edge_body(u_hbm, src_hbm, dst_hbm, ea_hbm, out_hbm,
               src2, dst2, ea2, rows, msg, buf, agg,
               isem, gsem, ssem):
    cid = lax.axis_index("c")
    sid = lax.axis_index("s")

    zero = jnp.zeros((H,), jnp.float32)

    def _zbody(i, c):
        buf[i, :] = zero
        return c
    lax.fori_loop(0, ROWS_PER_TILE, _zbody, 0, unroll=8)

    pltpu.sync_copy(buf, agg.at[pl.ds(sid * ROWS_PER_TILE, ROWS_PER_TILE)])
    plsc.subcore_barrier()

    wid = sid * NC + cid
    wbase = wid * EW

    def stage(p):
        b = p % 2
        base = wbase + p * PAIR
        n = _PLEN[p]
        return [
            pltpu.async_copy(src_hbm.at[pl.ds(base, n)],
                             src2.at[b, pl.ds(0, n)], isem),
            pltpu.async_copy(dst_hbm.at[pl.ds(base, n)],
                             dst2.at[b, pl.ds(0, n)], isem),
            pltpu.async_copy(ea_hbm.at[pl.ds(base * D_E, n * D_E)],
                             ea2.at[b, pl.ds(0, n * D_E)], isem),
        ]

    def gather(h):
        p, hh, hb, n = h // 2, h % 2, h % 2, _HLEN[h]
        return pltpu.async_copy(
            u_hbm.at[src2.at[p % 2, pl.ds(hh * n, n)]],
            rows.at[hb, pl.ds(0, n)], gsem)

    def compute(h):
        hb = h % 2
        p, hh, b, n = h // 2, h % 2, (h // 2) % 2, _HLEN[h]
        ea_off = hh * n * D_E

        def _gbody(g, c):
            a = ea2[b, pl.ds(ea_off + H * g, H)]
            for q in range(4):
                e = 4 * g + q
                m = rows[hb, e, pl.ds(4 * H, H)]
                m = m + a[4 * q + 0] * rows[hb, e, pl.ds(0, H)]
                m = m + a[4 * q + 1] * rows[hb, e, pl.ds(H, H)]
                m = m + a[4 * q + 2] * rows[hb, e, pl.ds(2 * H, H)]
                m = m + a[4 * q + 3] * rows[hb, e, pl.ds(3 * H, H)]
                msg[hb, e, :] = m
            return c
        lax.fori_loop(0, n // 4, _gbody, 0, unroll=2)

    def scatter(h):
        hb = h % 2
        p, hh, b, n = h // 2, h % 2, (h // 2) % 2, _HLEN[h]
        idx = dst2.at[b, pl.ds(hh * n, n)]
        return pltpu.async_copy(msg.at[hb, pl.ds(0, n)], agg.at[idx], ssem,
                                add=True)

    st = {0: stage(0)}
    for d in st[0]:
        d.wait()
    gd = {0: gather(0)}
    sc = {}
    for h in range(NH):
        p = h // 2
        if h >= 1:
            sc[h - 1].wait()
        if h + 1 < NH:
            if (h + 1) % 2 == 1:
                gd[h + 1] = gather(h + 1)
                if p + 1 < NPAIRS:
                    st[p + 1] = stage(p + 1)
            else:
                for d in st[p + 1]:
                    d.wait()
                gd[h + 1] = gather(h + 1)
        gd[h].wait()
        compute(h)
        sc[h] = scatter(h)
    sc[NH - 1].wait()

    plsc.subcore_barrier()
    pltpu.sync_copy(agg.at[pl.ds(sid * ROWS_PER_TILE, ROWS_PER_TILE)], buf)
    pltpu.sync_copy(buf,
                    out_hbm.at[cid, pl.ds(sid * ROWS_PER_TILE,
                                          ROWS_PER_TILE)])


_edge_pass = pl.kernel(
    _edge_body,
    out_type=jax.ShapeDtypeStruct((NC, NPAD, H), jnp.float32),
    mesh=plsc.VectorSubcoreMesh(core_axis_name="c", subcore_axis_name="s",
                                num_cores=NC, num_subcores=NS),
    compiler_params=pltpu.CompilerParams(use_tc_tiling_on_sc=False),
    scratch_types=[
        pltpu.VMEM((2, PAIR), jnp.int32),
        pltpu.VMEM((2, PAIR), jnp.int32),
        pltpu.VMEM((2, PAIR * D_E + H), jnp.float32),
        pltpu.VMEM((2, CH, UROW), jnp.float32),
        pltpu.VMEM((2, CH, H), jnp.float32),
        pltpu.VMEM((ROWS_PER_TILE, H), jnp.float32),
        pltpu.VMEM_SHARED((NPAD, H), jnp.float32),
        pltpu.SemaphoreType.DMA,
        pltpu.SemaphoreType.DMA,
        pltpu.SemaphoreType.DMA,
    ],
)


def kernel(x, edge_index, edge_attr, lin_in_w, lin_in_b, enn_w, enn_b,
           root_w, root_b, bn_g, bn_b, cls_w, cls_b):
    src, dst, ea_flat = _edge_prep(edge_index, edge_attr)

    wcat = jnp.concatenate(
        [enn_w.reshape(L, D_E, H, H).transpose(0, 2, 1, 3).reshape(L, H, D_E * H),
         enn_b.reshape(L, H, H)], axis=2)

    u, hr = _in_proj(x, lin_in_w, lin_in_b.reshape(1, H),
                     wcat[0], root_w[0], root_b[0].reshape(1, H))
    for l in range(L):
        part = _edge_pass(u, src, dst, ea_flat)
        if l + 1 < L:
            u, hr = _mid(hr, part, bn_g[l].reshape(1, H), bn_b[l].reshape(1, H),
                         wcat[l + 1], root_w[l + 1], root_b[l + 1].reshape(1, H))
        else:
            out = _final(hr, part, bn_g[l].reshape(1, H), bn_b[l].reshape(1, H),
                         cls_w, cls_b.reshape(1, OUT))
    return out


# speedup vs baseline: 1.4137x; 1.0179x over previous
import functools

import jax
import jax.numpy as jnp
from jax import lax
from jax.experimental import pallas as pl
from jax.experimental.pallas import tpu as pltpu
from jax.experimental.pallas import tpu_sc as plsc

N = 10000
E = 320000
D_IN = 128
H = 16
D_E = 4
L = 3
OUT = 2

NC = 2
NS = 16
NW = NC * NS
EW = E // NW
PAIR = 1024
LASTP = EW - 9 * PAIR
NPAIRS = 10
CH = PAIR // 2
LASTH = LASTP // 2
NH = NPAIRS * 2
_PLEN = [PAIR] * 9 + [LASTP]
_HLEN = [CH] * 18 + [LASTH] * 2
UROW = (D_E + 1) * H
NPAD = 10112
ROWS_PER_TILE = NPAD // NS

_PREC = lax.Precision.HIGHEST


def _in_proj_kernel(x_ref, w_ref, b_ref, wcat_ref, rw_ref, rb_ref,
                    u_ref, hr_ref):
    h0 = jnp.dot(x_ref[...], w_ref[...], preferred_element_type=jnp.float32,
                 precision=_PREC) + b_ref[...]
    u_ref[...] = jnp.dot(h0, wcat_ref[...],
                         preferred_element_type=jnp.float32, precision=_PREC)
    hr_ref[...] = jnp.dot(h0, rw_ref[...], preferred_element_type=jnp.float32,
                          precision=_PREC) + rb_ref[...]


def _mid_kernel(hr_ref, part_ref, g_ref, b_ref, wcat_ref, rw_ref, rb_ref,
                u_ref, hro_ref):
    t = (hr_ref[...] + part_ref[0, pl.ds(0, N), :]
         + part_ref[1, pl.ds(0, N), :])
    mean = jnp.mean(t, axis=0, keepdims=True)
    var = jnp.mean((t - mean) ** 2, axis=0, keepdims=True)
    hn = (t - mean) * lax.rsqrt(var + 1e-5) * g_ref[...] + b_ref[...]
    hn = jnp.maximum(hn, 0.0)
    u_ref[...] = jnp.dot(hn, wcat_ref[...],
                         preferred_element_type=jnp.float32, precision=_PREC)
    hro_ref[...] = jnp.dot(hn, rw_ref[...], preferred_element_type=jnp.float32,
                           precision=_PREC) + rb_ref[...]


def _final_kernel(hr_ref, part_ref, g_ref, b_ref, cw_ref, cb_ref, o_ref):
    t = (hr_ref[...] + part_ref[0, pl.ds(0, N), :]
         + part_ref[1, pl.ds(0, N), :])
    mean = jnp.mean(t, axis=0, keepdims=True)
    var = jnp.mean((t - mean) ** 2, axis=0, keepdims=True)
    hn = (t - mean) * lax.rsqrt(var + 1e-5) * g_ref[...] + b_ref[...]
    hn = jnp.maximum(hn, 0.0)
    o_ref[...] = jnp.dot(hn, cw_ref[...], preferred_element_type=jnp.float32,
                         precision=_PREC) + cb_ref[...]


_in_proj = pl.pallas_call(
    _in_proj_kernel,
    out_shape=(jax.ShapeDtypeStruct((N, UROW), jnp.float32),
               jax.ShapeDtypeStruct((N, H), jnp.float32)),
)

_mid = pl.pallas_call(
    _mid_kernel,
    out_shape=(jax.ShapeDtypeStruct((N, UROW), jnp.float32),
               jax.ShapeDtypeStruct((N, H), jnp.float32)),
)

_final = pl.pallas_call(
    _final_kernel,
    out_shape=jax.ShapeDtypeStruct((N, OUT), jnp.float32),
)


PCH = 1280
NPCH = E // PCH
SUB = 256


def _prep_body(ei_hbm, ea_hbm, src_out, dst_out, eaf_out,
               idxb, eab, outb, sem):
    cid = lax.axis_index("c")
    sid = lax.axis_index("s")
    wid = sid * NC + cid
    iot = lax.iota(jnp.int32, 16)
    ri0 = lax.shift_right_logical(iot, 2)
    li = lax.bitwise_and(iot, 3)
    for k in range(8):
        c = wid + NW * k

        @pl.when(c < NPCH)
        def _():
            base = c * PCH
            pltpu.sync_copy(ei_hbm.at[0, pl.ds(base, PCH)], idxb)
            pltpu.sync_copy(idxb, src_out.at[pl.ds(base, PCH)])
            pltpu.sync_copy(ei_hbm.at[1, pl.ds(base, PCH)], idxb)
            pltpu.sync_copy(idxb, dst_out.at[pl.ds(base, PCH)])
            for j in range(PCH // SUB):
                pltpu.sync_copy(ea_hbm.at[pl.ds(base + j * SUB, SUB)], eab)

                def _gbody(g, carry, _j=j):
                    v = plsc.load_gather(eab, [ri0 + 4 * g, li])
                    outb[pl.ds(_j * SUB * D_E + 16 * g, 16)] = v
                    return carry
                lax.fori_loop(0, SUB // 4, _gbody, 0, unroll=4)
            pltpu.sync_copy(outb, eaf_out.at[pl.ds(base * D_E, PCH * D_E)])


_edge_prep = pl.kernel(
    _prep_body,
    out_type=(jax.ShapeDtypeStruct((E,), jnp.int32),
              jax.ShapeDtypeStruct((E,), jnp.int32),
              jax.ShapeDtypeStruct((E * D_E,), jnp.float32)),
    mesh=plsc.VectorSubcoreMesh(core_axis_name="c", subcore_axis_name="s",
                                num_cores=NC, num_subcores=NS),
    compiler_params=pltpu.CompilerParams(needs_layout_passes=False),
    scratch_types=[
        pltpu.VMEM((PCH,), jnp.int32),
        pltpu.VMEM((SUB, D_E), jnp.float32),
        pltpu.VMEM((PCH * D_E,), jnp.float32),
        pltpu.SemaphoreType.DMA,
    ],
)


def _edge_body(u_hbm, src_hbm, dst_hbm, ea_hbm, out_hbm,
               src2, dst2, ea2, rows, msg, buf, agg,
               isem, gsem, ssem):
    cid = lax.axis_index("c")
    sid = lax.axis_index("s")

    zero = jnp.zeros((H,), jnp.float32)

    def _zbody(i, c):
        buf[i, :] = zero
        return c
    lax.fori_loop(0, ROWS_PER_TILE, _zbody, 0, unroll=8)

    pltpu.sync_copy(buf, agg.at[pl.ds(sid * ROWS_PER_TILE, ROWS_PER_TILE)])
    plsc.subcore_barrier()

    wid = sid * NC + cid
    wbase = wid * EW

    def stage(p):
        b = p % 2
        base = wbase + p * PAIR
        n = _PLEN[p]
        return [
            pltpu.async_copy(src_hbm.at[pl.ds(base, n)],
                             src2.at[b, pl.ds(0, n)], isem),
            pltpu.async_copy(dst_hbm.at[pl.ds(base, n)],
                             dst2.at[b, pl.ds(0, n)], isem),
            pltpu.async_copy(ea_hbm.at[pl.ds(base * D_E, n * D_E)],
                             ea2.at[b, pl.ds(0, n * D_E)], isem),
        ]

    def gather(h):
        p, hh, hb, n = h // 2, h % 2, h % 2, _HLEN[h]
        return pltpu.async_copy(
            u_hbm.at[src2.at[p % 2, pl.ds(hh * n, n)]],
            rows.at[hb, pl.ds(0, n)], gsem)

    def compute(h):
        hb = h % 2
        p, hh, b, n = h // 2, h % 2, (h // 2) % 2, _HLEN[h]
        ea_off = hh * n * D_E

        def _gbody(g, c):
            a = ea2[b, pl.ds(ea_off + H * g, H)]
            for q in range(4):
                e = 4 * g + q
                m = rows[hb, e, pl.ds(4 * H, H)]
                m = m + a[4 * q + 0] * rows[hb, e, pl.ds(0, H)]
                m = m + a[4 * q + 1] * rows[hb, e, pl.ds(H, H)]
                m = m + a[4 * q + 2] * rows[hb, e, pl.ds(2 * H, H)]
                m = m + a[4 * q + 3] * rows[hb, e, pl.ds(3 * H, H)]
                msg[hb, e, :] = m
            return c
        lax.fori_loop(0, n // 4, _gbody, 0, unroll=2)

    def scatter(h):
        hb = h % 2
        p, hh, b, n = h // 2, h % 2, (h // 2) % 2, _HLEN[h]
        idx = dst2.at[b, pl.ds(hh * n, n)]
        return pltpu.async_copy(msg.at[hb, pl.ds(0, n)], agg.at[idx], ssem,
                                add=True)

    st = {0: stage(0)}
    for d in st[0]:
        d.wait()
    gd = {0: gather(0)}
    sc = {}
    waited = set()

    def drain(i):
        if 0 <= i < NH and i not in waited and i in sc:
            sc[i].wait()
            waited.add(i)

    for h in range(NH):
        p = h // 2
        drain(h - 2)
        if h + 1 < NH:
            if (h + 1) % 2 == 1:
                gd[h + 1] = gather(h + 1)
                if p + 1 < NPAIRS:
                    drain(h - 1)
                    st[p + 1] = stage(p + 1)
            else:
                for d in st[p + 1]:
                    d.wait()
                gd[h + 1] = gather(h + 1)
        gd[h].wait()
        compute(h)
        sc[h] = scatter(h)
    drain(NH - 2)
    drain(NH - 1)

    plsc.subcore_barrier()
    pltpu.sync_copy(agg.at[pl.ds(sid * ROWS_PER_TILE, ROWS_PER_TILE)], buf)
    pltpu.sync_copy(buf,
                    out_hbm.at[cid, pl.ds(sid * ROWS_PER_TILE,
                                          ROWS_PER_TILE)])


_edge_pass = pl.kernel(
    _edge_body,
    out_type=jax.ShapeDtypeStruct((NC, NPAD, H), jnp.float32),
    mesh=plsc.VectorSubcoreMesh(core_axis_name="c", subcore_axis_name="s",
                                num_cores=NC, num_subcores=NS),
    compiler_params=pltpu.CompilerParams(use_tc_tiling_on_sc=False),
    scratch_types=[
        pltpu.VMEM((2, PAIR), jnp.int32),
        pltpu.VMEM((2, PAIR), jnp.int32),
        pltpu.VMEM((2, PAIR * D_E + H), jnp.float32),
        pltpu.VMEM((2, CH, UROW), jnp.float32),
        pltpu.VMEM((2, CH, H), jnp.float32),
        pltpu.VMEM((ROWS_PER_TILE, H), jnp.float32),
        pltpu.VMEM_SHARED((NPAD, H), jnp.float32),
        pltpu.SemaphoreType.DMA,
        pltpu.SemaphoreType.DMA,
        pltpu.SemaphoreType.DMA,
    ],
)


def kernel(x, edge_index, edge_attr, lin_in_w, lin_in_b, enn_w, enn_b,
           root_w, root_b, bn_g, bn_b, cls_w, cls_b):
    src, dst, ea_flat = _edge_prep(edge_index, edge_attr)

    wcat = jnp.concatenate(
        [enn_w.reshape(L, D_E, H, H).transpose(0, 2, 1, 3).reshape(L, H, D_E * H),
         enn_b.reshape(L, H, H)], axis=2)

    u, hr = _in_proj(x, lin_in_w, lin_in_b.reshape(1, H),
                     wcat[0], root_w[0], root_b[0].reshape(1, H))
    for l in range(L):
        part = _edge_pass(u, src, dst, ea_flat)
        if l + 1 < L:
            u, hr = _mid(hr, part, bn_g[l].reshape(1, H), bn_b[l].reshape(1, H),
                         wcat[l + 1], root_w[l + 1], root_b[l + 1].reshape(1, H))
        else:
            out = _final(hr, part, bn_g[l].reshape(1, H), bn_b[l].reshape(1, H),
                         cls_w, cls_b.reshape(1, OUT))
    return out
